# baseline (speedup 1.0000x reference)
import functools
import math

import jax
import jax.numpy as jnp
from jax import lax
from jax.experimental import pallas as pl
from jax.experimental.pallas import tpu as pltpu
from jax.experimental.pallas import tpu_sc as plsc

NUM_TABLES = 26
VOCAB = 100000
EMB_DIM = 32
OFFSET = 13
B, S = 4096, 50
SCALE = math.sqrt(EMB_DIM)

NC, NS = 2, 16
NW = NC * NS
TB = 512
NCHUNK = B // TB
NTASK = S * NUM_TABLES * NCHUNK
TASKS_PER_W = NTASK // NW
D_EMB = NUM_TABLES * EMB_DIM
TILE = 16384
UNROLL = 16


def _build_sc_kernel():
    mesh = plsc.VectorSubcoreMesh(core_axis_name="c", subcore_axis_name="s")

    @functools.partial(
        pl.kernel,
        mesh=mesh,
        out_type=jax.ShapeDtypeStruct((S * D_EMB * B,), jnp.float32),
        compiler_params=pltpu.CompilerParams(
            use_tc_tiling_on_sc=False, needs_layout_passes=False
        ),
        scratch_types=[
            pltpu.VMEM((2 * TB,), jnp.int32),
            pltpu.VMEM((2 * TB, EMB_DIM), jnp.float32),
            pltpu.VMEM((2 * TILE,), jnp.float32),
            pltpu.SemaphoreType.DMA,
            pltpu.SemaphoreType.DMA,
        ],
    )
    def k(tables_hbm, idx_hbm, out_hbm, idx_v, stage, tb_v, sem_g, sem_o):
        wid = lax.axis_index("s") * NC + lax.axis_index("c")
        tid0 = wid * TASKS_PER_W
        lane = lax.iota(jnp.int32, 16)
        base_vec = [
            ((lane + 16 * h) // 8) * 4096 + lax.rem(lane + 16 * h, 8) * 128
            for h in (0, 1)
        ]

        def task_coords(tid):
            st = tid // NCHUNK
            c = tid % NCHUNK
            t = st % NUM_TABLES
            s = st // NUM_TABLES
            return s, t, c

        def issue_gathers(tid, sel):
            s, t, c = task_coords(tid)
            pltpu.sync_copy(
                idx_hbm.at[pl.ds((s * NUM_TABLES + t) * B + c * TB, TB)],
                idx_v.at[pl.ds(sel * TB, TB)],
            )
            for j in range(TB // 128):
                pltpu.async_copy(
                    tables_hbm.at[
                        idx_v.at[pl.ds(sel * TB + j * 128, 128)]
                    ],
                    stage.at[pl.ds(sel * TB + j * 128, 128)],
                    sem_g,
                )

        def drain_gather(sel):
            pltpu.make_async_copy(
                tables_hbm.at[pl.ds(0, TB)],
                stage.at[pl.ds(sel * TB, TB)],
                sem_g,
            ).wait()

        def drain_out():
            pltpu.make_async_copy(
                out_hbm.at[pl.ds(0, TILE)],
                tb_v.at[pl.ds(0, TILE)],
                sem_o,
            ).wait()

        def compute(sel):
            def rr_body(rr, rcarry):
                base_sc = (rr // 8) * 1024 + lax.rem(rr, 8) * UNROLL
                for u in range(UNROLL):
                    row = sel * TB + rr * UNROLL + u
                    bb = jnp.full((16,), sel * TILE + base_sc + u, jnp.int32)
                    for h in (0, 1):
                        v = stage[row, pl.ds(16 * h, 16)] * SCALE
                        plsc.store_scatter(tb_v, [bb + base_vec[h]], v)
                return rcarry

            lax.fori_loop(0, TB // UNROLL, rr_body, 0)

        def out_dma(tid, sel):
            s, t, c = task_coords(tid)
            dst_base = ((s * (D_EMB // 8) + 4 * t) * 32 + 4 * c) * 1024
            for i in range(4):
                for bt in range(4):
                    pltpu.async_copy(
                        tb_v.at[pl.ds(sel * TILE + i * 4096 + bt * 1024, 1024)],
                        out_hbm.at[pl.ds(dst_base + i * 32768 + bt * 1024, 1024)],
                        sem_o,
                    )

        issue_gathers(tid0, 0)

        def pair_body(m, carry):
            k0 = 2 * m
            issue_gathers(tid0 + k0 + 1, 1)
            drain_gather(0)

            @pl.when(m >= 1)
            def _():
                drain_out()

            compute(0)
            out_dma(tid0 + k0, 0)
            issue_gathers(tid0 + k0 + 2, 0)
            drain_gather(1)

            @pl.when(m >= 1)
            def _():
                drain_out()

            compute(1)
            out_dma(tid0 + k0 + 1, 1)
            return carry

        lax.fori_loop(0, (TASKS_PER_W - 1) // 2, pair_body, 0)

        drain_gather(0)
        drain_out()
        compute(0)
        out_dma(tid0 + TASKS_PER_W - 1, 0)
        drain_out()
        drain_out()

    return k


def kernel(x, tables):
    cont_t = jnp.transpose(x[:, :, :OFFSET], (1, 2, 0))
    offs = (jnp.arange(NUM_TABLES, dtype=jnp.int32) * VOCAB)[None, :, None]
    idx_t = jnp.transpose(x[:, :, OFFSET:].astype(jnp.int32), (1, 2, 0)) + offs
    idx = idx_t.reshape(-1)
    tflat = tables.reshape(NUM_TABLES * VOCAB, EMB_DIM)
    emb = _build_sc_kernel()(tflat, idx)
    emb5 = emb.reshape(S, D_EMB // 8, B // 128, 8, 128)
    emb_t = jnp.transpose(emb5, (0, 1, 3, 2, 4)).reshape(S, D_EMB, B)
    full = jnp.concatenate([cont_t, emb_t], axis=1)
    return jnp.transpose(full, (2, 0, 1))


# speedup vs baseline: 1.5334x; 1.5334x over previous
import functools
import math

import jax
import jax.numpy as jnp
from jax import lax
from jax.experimental import pallas as pl
from jax.experimental.pallas import tpu as pltpu
from jax.experimental.pallas import tpu_sc as plsc

NUM_TABLES = 26
VOCAB = 100000
EMB_DIM = 32
OFFSET = 13
B, S = 4096, 50
SCALE = math.sqrt(EMB_DIM)

NC, NS = 2, 16
NW = NC * NS
TB = 512
NCHUNK = B // TB
NTASK = S * NUM_TABLES * NCHUNK
TASKS_PER_W = NTASK // NW
D_EMB = NUM_TABLES * EMB_DIM
UNROLL = 16
PITCH = 1041
SELW = 520


def _build_sc_kernel():
    mesh = plsc.VectorSubcoreMesh(core_axis_name="c", subcore_axis_name="s")

    @functools.partial(
        pl.kernel,
        mesh=mesh,
        out_type=jax.ShapeDtypeStruct((S * D_EMB * B // 128, 128), jnp.float32),
        compiler_params=pltpu.CompilerParams(
            use_tc_tiling_on_sc=False, needs_layout_passes=False
        ),
        scratch_types=[
            pltpu.VMEM((2 * TB,), jnp.int32),
            pltpu.VMEM((2 * TB, EMB_DIM), jnp.float32),
            pltpu.VMEM((EMB_DIM, PITCH), jnp.float32),
            pltpu.SemaphoreType.DMA,
            pltpu.SemaphoreType.DMA,
        ],
    )
    def k(tables_hbm, idx_hbm, out_hbm, idx_v, stage, tb_v, sem_g, sem_o):
        wid = lax.axis_index("s") * NC + lax.axis_index("c")
        tid0 = wid * TASKS_PER_W
        lane = lax.iota(jnp.int32, 16)
        e_idx = [lane + 16 * h for h in (0, 1)]

        def task_coords(tid):
            st = tid // NCHUNK
            c = tid % NCHUNK
            t = st % NUM_TABLES
            s = st // NUM_TABLES
            return s, t, c

        def issue_gathers(tid, sel):
            s, t, c = task_coords(tid)
            pltpu.sync_copy(
                idx_hbm.at[pl.ds((s * NUM_TABLES + t) * B + c * TB, TB)],
                idx_v.at[pl.ds(sel * TB, TB)],
            )
            for j in range(TB // 128):
                pltpu.async_copy(
                    tables_hbm.at[
                        idx_v.at[pl.ds(sel * TB + j * 128, 128)]
                    ],
                    stage.at[pl.ds(sel * TB + j * 128, 128)],
                    sem_g,
                )

        def drain_gather(sel):
            pltpu.make_async_copy(
                tables_hbm.at[pl.ds(0, TB)],
                stage.at[pl.ds(sel * TB, TB)],
                sem_g,
            ).wait()

        def drain_out():
            for i in range(4):
                for bt in range(4):
                    pltpu.make_async_copy(
                        out_hbm.at[pl.ds(0, 8), :],
                        tb_v.at[pl.ds(8 * i, 8), pl.ds(128 * bt, 128)],
                        sem_o,
                    ).wait()

        def compute(sel):
            def rr_body(rr, rcarry):
                for u in range(UNROLL):
                    row = sel * TB + rr * UNROLL + u
                    bb = jnp.full(
                        (16,), sel * SELW + rr * UNROLL + u, jnp.int32
                    )
                    for h in (0, 1):
                        v = stage[row, pl.ds(16 * h, 16)] * SCALE
                        plsc.store_scatter(tb_v, [e_idx[h], bb], v)
                return rcarry

            lax.fori_loop(0, TB // UNROLL, rr_body, 0)

        def out_dma(tid, sel):
            s, t, c = task_coords(tid)
            dst_row = ((s * (D_EMB // 8) + 4 * t) * 32 + 4 * c) * 8
            for i in range(4):
                for bt in range(4):
                    pltpu.async_copy(
                        tb_v.at[
                            pl.ds(8 * i, 8),
                            pl.ds(sel * SELW + 128 * bt, 128),
                        ],
                        out_hbm.at[pl.ds(dst_row + i * 256 + bt * 8, 8), :],
                        sem_o,
                    )

        issue_gathers(tid0, 0)

        def pair_body(m, carry):
            k0 = 2 * m
            issue_gathers(tid0 + k0 + 1, 1)
            drain_gather(0)

            @pl.when(m >= 1)
            def _():
                drain_out()

            compute(0)
            out_dma(tid0 + k0, 0)
            issue_gathers(tid0 + k0 + 2, 0)
            drain_gather(1)

            @pl.when(m >= 1)
            def _():
                drain_out()

            compute(1)
            out_dma(tid0 + k0 + 1, 1)
            return carry

        lax.fori_loop(0, (TASKS_PER_W - 1) // 2, pair_body, 0)

        drain_gather(0)
        drain_out()
        compute(0)
        out_dma(tid0 + TASKS_PER_W - 1, 0)
        drain_out()
        drain_out()

    return k


def kernel(x, tables):
    cont_t = jnp.transpose(x[:, :, :OFFSET], (1, 2, 0))
    offs = (jnp.arange(NUM_TABLES, dtype=jnp.int32) * VOCAB)[None, :, None]
    idx_t = jnp.transpose(x[:, :, OFFSET:].astype(jnp.int32), (1, 2, 0)) + offs
    idx = idx_t.reshape(-1)
    tflat = tables.reshape(NUM_TABLES * VOCAB, EMB_DIM)
    emb = _build_sc_kernel()(tflat, idx)
    emb5 = emb.reshape(S, D_EMB // 8, B // 128, 8, 128)
    emb_t = jnp.transpose(emb5, (0, 1, 3, 2, 4)).reshape(S, D_EMB, B)
    full = jnp.concatenate([cont_t, emb_t], axis=1)
    return jnp.transpose(full, (2, 0, 1))
